# baseline (speedup 1.0000x reference)
import functools

import jax
import jax.numpy as jnp
from jax import lax
from jax.experimental import pallas as pl
from jax.experimental.pallas import tpu as pltpu
from jax.experimental.pallas import tpu_sc as plsc

V = 1000000
D = 16
B = 16384
L = 50
N = B * L

NC = 2
NS = 16
NW = NC * NS
PER_W = N // NW
C = 1600
NCH = PER_W // C


def _dequant_lookup(idx_hbm, w_hbm, scale_hbm, out_hbm,
                    idx_v, idxq_v, idxs_v, rows_v, scale_v, out_v, sem):
    wid = lax.axis_index("s") * NC + lax.axis_index("c")
    wbase = wid * PER_W

    iota = lax.iota(jnp.int32, 16)
    qiota = iota >> 2
    riota = iota & 3
    siota = iota * 4

    def recidx(m, carry):
        val = idx_v[pl.ds(m * 16, 16)]
        idxq_v[pl.ds(m * 16, 16)] = val >> 2
        idxs_v[pl.ds(m * 16, 16)] = val >> 4
        return carry

    def body(k, carry):
        look = qiota + 4 * k
        ql = plsc.load_gather(idx_v, [look])
        sub = ((ql & 3) << 2) + riota
        v = plsc.load_gather(rows_v, [look, sub])
        s = plsc.load_gather(scale_v, [look, ql & 15])
        base = k * 64
        b0 = ((v << 24) >> 24).astype(jnp.float32) * s
        b1 = ((v << 16) >> 24).astype(jnp.float32) * s
        b2 = ((v << 8) >> 24).astype(jnp.float32) * s
        b3 = (v >> 24).astype(jnp.float32) * s
        plsc.store_scatter(out_v, [siota + base], b0)
        plsc.store_scatter(out_v, [siota + (base + 1)], b1)
        plsc.store_scatter(out_v, [siota + (base + 2)], b2)
        plsc.store_scatter(out_v, [siota + (base + 3)], b3)
        return carry

    def chunk(ch, carry):
        base = wbase + ch * C
        pltpu.sync_copy(idx_hbm.at[pl.ds(base, C)], idx_v)
        lax.fori_loop(0, C // 16, recidx, 0, unroll=2)
        rows_dma = pltpu.async_copy(w_hbm.at[idxq_v], rows_v, sem)
        scale_dma = pltpu.async_copy(scale_hbm.at[idxs_v], scale_v, sem)
        rows_dma.wait()
        scale_dma.wait()
        lax.fori_loop(0, C * D // 64, body, 0, unroll=2)
        pltpu.sync_copy(out_v, out_hbm.at[pl.ds(base * D, C * D)])
        return carry

    lax.fori_loop(0, NCH, chunk, 0)


@jax.jit
def _run(idx, wq, scaleq):
    mesh = plsc.VectorSubcoreMesh(core_axis_name="c", subcore_axis_name="s")
    f = functools.partial(
        pl.kernel,
        mesh=mesh,
        out_type=jax.ShapeDtypeStruct((N * D,), jnp.float32),
        scratch_types=[
            pltpu.VMEM((C,), jnp.int32),
            pltpu.VMEM((C,), jnp.int32),
            pltpu.VMEM((C,), jnp.int32),
            pltpu.VMEM((C, 16), jnp.int32),
            pltpu.VMEM((C, 16), jnp.float32),
            pltpu.VMEM((C * D,), jnp.float32),
            pltpu.SemaphoreType.DMA,
        ],
        compiler_params=pltpu.CompilerParams(
            needs_layout_passes=False, use_tc_tiling_on_sc=False),
    )(_dequant_lookup)
    return f(idx, wq, scaleq)


def kernel(input, weight, weight_scale):
    idx = input.reshape(-1)
    wq = lax.bitcast_convert_type(weight.reshape(V // 4, 16, 4), jnp.int32)
    scaleq = weight_scale.reshape(V // 16, 16)
    out = _run(idx, wq, scaleq)
    return out.reshape(B, L, D)


# speedup vs baseline: 4.6950x; 4.6950x over previous
import functools

import jax
import jax.numpy as jnp
from jax import lax
from jax.experimental import pallas as pl
from jax.experimental.pallas import tpu as pltpu
from jax.experimental.pallas import tpu_sc as plsc

V = 1000000
D = 16
B = 16384
L = 50
N = B * L

NC = 2
NS = 16
NW = NC * NS
PER_W = N // NW
C = 1600
NCH = PER_W // C


def _dequant_lookup(idx_hbm, w_hbm, scale_hbm, out_hbm,
                    idx_v, idxq_v, idxs_v, rows_v, scale_v,
                    sub_v, scale_c, out_v, sem):
    wid = lax.axis_index("s") * NC + lax.axis_index("c")
    wbase = wid * PER_W

    wrec_hbm = w_hbm

    iota = lax.iota(jnp.int32, 16)
    qiota = iota >> 2
    riota = iota & 3
    lsh = 24 - riota * 8

    def recidx(m, carry):
        val = idx_v[pl.ds(m * 16, 16)]
        idxq_v[pl.ds(m * 16, 16)] = val >> 2
        idxs_v[pl.ds(m * 16, 16)] = val >> 4
        sub_v[pl.ds(m * 16, 16)] = (val & 3) << 2
        return carry

    def scalesel(m, carry):
        val = idx_v[pl.ds(m * 16, 16)]
        scale_c[pl.ds(m * 16, 16)] = plsc.load_gather(
            scale_v, [iota + m * 16, val & 15])
        return carry

    gdims = lax.GatherDimensionNumbers(
        offset_dims=(), collapsed_slice_dims=(0,), start_index_map=(0,))

    def one(k):
        rec = rows_v[k]
        rec32 = plsc.bitcast(rec, jnp.int32)
        sel = plsc.load_gather(sub_v, [iota * 0 + k]) + qiota
        w = lax.gather(rec32, sel[:, None], gdims, (1,),
                       mode=lax.GatherScatterMode.PROMISE_IN_BOUNDS)
        s = plsc.load_gather(scale_c, [iota * 0 + k])
        val = ((w << lsh) >> 24).astype(jnp.float32) * s
        out_v[pl.ds(k * 16, 16)] = val

    def body(k2, carry):
        one(k2 * 2)
        one(k2 * 2 + 1)
        return carry

    def chunk(ch, carry):
        base = wbase + ch * C
        pltpu.sync_copy(idx_hbm.at[pl.ds(base, C)], idx_v)
        rows_dma = None
        lax.fori_loop(0, C // 16, recidx, 0, unroll=2)
        rows_dma = pltpu.async_copy(wrec_hbm.at[idxq_v], rows_v, sem)
        scale_dma = pltpu.async_copy(scale_hbm.at[idxs_v], scale_v, sem)
        rows_dma.wait()
        scale_dma.wait()
        lax.fori_loop(0, C // 16, scalesel, 0, unroll=2)
        lax.fori_loop(0, C // 2, body, 0)
        pltpu.sync_copy(out_v, out_hbm.at[pl.ds(base * D, C * D)])
        return carry

    lax.fori_loop(0, NCH, chunk, 0)


@jax.jit
def _run(idx, weight, scaleq):
    mesh = plsc.VectorSubcoreMesh(core_axis_name="c", subcore_axis_name="s")
    f = functools.partial(
        pl.kernel,
        mesh=mesh,
        out_type=jax.ShapeDtypeStruct((N * D,), jnp.float32),
        scratch_types=[
            pltpu.VMEM((C,), jnp.int32),
            pltpu.VMEM((C,), jnp.int32),
            pltpu.VMEM((C,), jnp.int32),
            pltpu.VMEM((C, 64), jnp.int8),
            pltpu.VMEM((C, 16), jnp.float32),
            pltpu.VMEM((C,), jnp.int32),
            pltpu.VMEM((C,), jnp.float32),
            pltpu.VMEM((C * D,), jnp.float32),
            pltpu.SemaphoreType.DMA,
        ],
        compiler_params=pltpu.CompilerParams(
            needs_layout_passes=False, use_tc_tiling_on_sc=False),
    )(_dequant_lookup)
    return f(idx, weight, scaleq)


def kernel(input, weight, weight_scale):
    idx = input.reshape(-1)
    wrec = weight.reshape(V // 4, 64)
    scaleq = weight_scale.reshape(V // 16, 16)
    out = _run(idx, wrec, scaleq)
    return out.reshape(B, L, D)


# speedup vs baseline: 4.7098x; 1.0032x over previous
import functools

import jax
import jax.numpy as jnp
from jax import lax
from jax.experimental import pallas as pl
from jax.experimental.pallas import tpu as pltpu
from jax.experimental.pallas import tpu_sc as plsc

V = 1000000
D = 16
B = 16384
L = 50
N = B * L

NC = 2
NS = 16
NW = NC * NS
PER_W = N // NW
C = 1600
NCH = PER_W // C


def _dequant_lookup(idx_hbm, w_hbm, scale_hbm, out_hbm,
                    idx_v, idxq_v, idxs_v, rows_v, scale_v,
                    sub_v, scale_c, out_v, sem):
    wid = lax.axis_index("s") * NC + lax.axis_index("c")
    wbase = wid * PER_W

    wrec_hbm = w_hbm

    iota = lax.iota(jnp.int32, 16)
    qiota = iota >> 2
    riota = iota & 3
    lsh = 24 - riota * 8

    def recidx(m, carry):
        val = idx_v[pl.ds(m * 16, 16)]
        idxq_v[pl.ds(m * 16, 16)] = val >> 2
        idxs_v[pl.ds(m * 16, 16)] = val >> 4
        sub_v[pl.ds(m * 16, 16)] = (val & 3) << 2
        return carry

    def scalesel(m, carry):
        val = idx_v[pl.ds(m * 16, 16)]
        scale_c[pl.ds(m * 16, 16)] = plsc.load_gather(
            scale_v, [iota + m * 16, val & 15])
        return carry

    gdims = lax.GatherDimensionNumbers(
        offset_dims=(), collapsed_slice_dims=(0,), start_index_map=(0,))

    def one(k):
        rec = rows_v[k]
        rec32 = plsc.bitcast(rec, jnp.int32)
        sel = plsc.load_gather(sub_v, [iota * 0 + k]) + qiota
        w = lax.gather(rec32, sel[:, None], gdims, (1,),
                       mode=lax.GatherScatterMode.PROMISE_IN_BOUNDS)
        s = plsc.load_gather(scale_c, [iota * 0 + k])
        val = ((w << lsh) >> 24).astype(jnp.float32) * s
        out_v[pl.ds(k * 16, 16)] = val

    def body(k2, carry):
        one(k2 * 2)
        one(k2 * 2 + 1)
        return carry

    def chunk(ch, carry):
        base = wbase + ch * C
        pltpu.sync_copy(idx_hbm.at[pl.ds(base, C)], idx_v)
        rows_dma = None
        lax.fori_loop(0, C // 16, recidx, 0, unroll=2)
        rows_dma = pltpu.async_copy(wrec_hbm.at[idxq_v], rows_v, sem)
        scale_dma = pltpu.async_copy(scale_hbm.at[idxs_v], scale_v, sem)
        rows_dma.wait()
        scale_dma.wait()
        lax.fori_loop(0, C // 16, scalesel, 0, unroll=2)
        lax.fori_loop(0, C // 2, body, 0, unroll=2)
        pltpu.sync_copy(out_v, out_hbm.at[pl.ds(base * D, C * D)])
        return carry

    lax.fori_loop(0, NCH, chunk, 0)


@jax.jit
def _run(idx, weight, scaleq):
    mesh = plsc.VectorSubcoreMesh(core_axis_name="c", subcore_axis_name="s")
    f = functools.partial(
        pl.kernel,
        mesh=mesh,
        out_type=jax.ShapeDtypeStruct((N * D,), jnp.float32),
        scratch_types=[
            pltpu.VMEM((C,), jnp.int32),
            pltpu.VMEM((C,), jnp.int32),
            pltpu.VMEM((C,), jnp.int32),
            pltpu.VMEM((C, 64), jnp.int8),
            pltpu.VMEM((C, 16), jnp.float32),
            pltpu.VMEM((C,), jnp.int32),
            pltpu.VMEM((C,), jnp.float32),
            pltpu.VMEM((C * D,), jnp.float32),
            pltpu.SemaphoreType.DMA,
        ],
        compiler_params=pltpu.CompilerParams(
            needs_layout_passes=False, use_tc_tiling_on_sc=False),
    )(_dequant_lookup)
    return f(idx, weight, scaleq)


def kernel(input, weight, weight_scale):
    idx = input.reshape(-1)
    wrec = weight.reshape(V // 4, 64)
    scaleq = weight_scale.reshape(V // 16, 16)
    out = _run(idx, wrec, scaleq)
    return out.reshape(B, L, D)


# speedup vs baseline: 5.5684x; 1.1823x over previous
import functools

import jax
import jax.numpy as jnp
from jax import lax
from jax.experimental import pallas as pl
from jax.experimental.pallas import tpu as pltpu
from jax.experimental.pallas import tpu_sc as plsc

V = 1000000
D = 16
B = 16384
L = 50
N = B * L

NC = 2
NS = 16
NW = NC * NS
CB = 1024
NBC = B // CB
ITEMS = L * NBC
PER_W = ITEMS // NW


def _dequant_lookup(idx_hbm, w_hbm, scale_hbm, out_hbm,
                    idx_v, idxq_v, idxs_v, rows_v, scale_v,
                    sub_v, scale_c, out_v, sem):
    wid = lax.axis_index("s") * NC + lax.axis_index("c")

    iota = lax.iota(jnp.int32, 16)
    qiota = iota >> 2
    riota = iota & 3
    lsh = 24 - riota * 8
    plane = iota * CB

    gdims = lax.GatherDimensionNumbers(
        offset_dims=(), collapsed_slice_dims=(0,), start_index_map=(0,))

    def recidx(m, carry):
        val = idx_v[pl.ds(m * 16, 16)]
        idxq_v[pl.ds(m * 16, 16)] = val >> 2
        idxs_v[pl.ds(m * 16, 16)] = val >> 4
        sub_v[pl.ds(m * 16, 16)] = (val & 3) << 2
        return carry

    def scalesel(m, carry):
        val = idx_v[pl.ds(m * 16, 16)]
        scale_c[pl.ds(m * 16, 16)] = plsc.load_gather(
            scale_v, [iota + m * 16, val & 15])
        return carry

    def one(k):
        rec = rows_v[k]
        rec32 = plsc.bitcast(rec, jnp.int32)
        sel = plsc.load_gather(sub_v, [iota * 0 + k]) + qiota
        w = lax.gather(rec32, sel[:, None], gdims, (1,),
                       mode=lax.GatherScatterMode.PROMISE_IN_BOUNDS)
        s = plsc.load_gather(scale_c, [iota * 0 + k])
        val = ((w << lsh) >> 24).astype(jnp.float32) * s
        plsc.store_scatter(out_v, [iota, iota * 0 + k], val)

    def body(k2, carry):
        one(k2 * 2)
        one(k2 * 2 + 1)
        return carry

    def item_loop(t, carry):
        item = wid * PER_W + t
        lrow = item >> 4
        bc = item & 15
        base = lrow * B + bc * CB
        pltpu.sync_copy(idx_hbm.at[pl.ds(base, CB)], idx_v)
        lax.fori_loop(0, CB // 16, recidx, 0, unroll=2)
        rows_dma = pltpu.async_copy(w_hbm.at[idxq_v], rows_v, sem)
        scale_dma = pltpu.async_copy(scale_hbm.at[idxs_v], scale_v, sem)
        rows_dma.wait()
        scale_dma.wait()
        lax.fori_loop(0, CB // 16, scalesel, 0, unroll=2)
        lax.fori_loop(0, CB // 2, body, 0, unroll=2)
        pltpu.sync_copy(
            out_v, out_hbm.at[pl.ds(lrow * D, D), pl.ds(bc * CB, CB)])
        return carry

    lax.fori_loop(0, PER_W, item_loop, 0)


@jax.jit
def _run(idxt, weight, scaleq):
    mesh = plsc.VectorSubcoreMesh(core_axis_name="c", subcore_axis_name="s")
    f = functools.partial(
        pl.kernel,
        mesh=mesh,
        out_type=jax.ShapeDtypeStruct((L * D, B), jnp.float32),
        scratch_types=[
            pltpu.VMEM((CB,), jnp.int32),
            pltpu.VMEM((CB,), jnp.int32),
            pltpu.VMEM((CB,), jnp.int32),
            pltpu.VMEM((CB, 64), jnp.int8),
            pltpu.VMEM((CB, 16), jnp.float32),
            pltpu.VMEM((CB,), jnp.int32),
            pltpu.VMEM((CB,), jnp.float32),
            pltpu.VMEM((D, CB), jnp.float32),
            pltpu.SemaphoreType.DMA,
        ],
        compiler_params=pltpu.CompilerParams(
            needs_layout_passes=False, use_tc_tiling_on_sc=False),
    )(_dequant_lookup)
    return f(idxt, weight, scaleq)


def kernel(input, weight, weight_scale):
    idxt = input.T.reshape(-1)
    wrec = weight.reshape(V // 4, 64)
    scaleq = weight_scale.reshape(V // 16, 16)
    out = _run(idxt, wrec, scaleq)
    return out.reshape(L, D, B).transpose(2, 0, 1)


# speedup vs baseline: 5.6386x; 1.0126x over previous
import functools

import jax
import jax.numpy as jnp
from jax import lax
from jax.experimental import pallas as pl
from jax.experimental.pallas import tpu as pltpu
from jax.experimental.pallas import tpu_sc as plsc

V = 1000000
D = 16
B = 16384
L = 50
N = B * L

NC = 2
NS = 16
NW = NC * NS
CB = 1024
NBC = B // CB
ITEMS = L * NBC
PER_W = ITEMS // NW


def _dequant_lookup(idx_hbm, w_hbm, scale_hbm, out_hbm,
                    idx_v, idxq_v, idxs_v, rows_v, scale_v,
                    sub_v, scale_c, out_v, sem, osem):
    wid = lax.axis_index("s") * NC + lax.axis_index("c")

    iota = lax.iota(jnp.int32, 16)
    qiota = iota >> 2
    riota = iota & 3
    lsh = 24 - riota * 8
    plane = iota * CB

    gdims = lax.GatherDimensionNumbers(
        offset_dims=(), collapsed_slice_dims=(0,), start_index_map=(0,))

    def recidx(m, carry):
        val = idx_v[pl.ds(m * 16, 16)]
        idxq_v[pl.ds(m * 16, 16)] = val >> 2
        idxs_v[pl.ds(m * 16, 16)] = val >> 4
        sub_v[pl.ds(m * 16, 16)] = (val & 3) << 2
        return carry

    def scalesel(m, carry):
        val = idx_v[pl.ds(m * 16, 16)]
        scale_c[pl.ds(m * 16, 16)] = plsc.load_gather(
            scale_v, [iota + m * 16, val & 15])
        return carry

    def one(k):
        rec = rows_v[k]
        rec32 = plsc.bitcast(rec, jnp.int32)
        sel = plsc.load_gather(sub_v, [iota * 0 + k]) + qiota
        w = lax.gather(rec32, sel[:, None], gdims, (1,),
                       mode=lax.GatherScatterMode.PROMISE_IN_BOUNDS)
        s = plsc.load_gather(scale_c, [iota * 0 + k])
        val = ((w << lsh) >> 24).astype(jnp.float32) * s
        plsc.store_scatter(out_v, [iota, iota * 0 + k], val)

    def body(k2, carry):
        one(k2 * 2)
        one(k2 * 2 + 1)
        return carry

    def item_loop(t, carry):
        item = wid * PER_W + t
        lrow = item >> 4
        bc = item & 15
        base = lrow * B + bc * CB
        pltpu.sync_copy(idx_hbm.at[pl.ds(base, CB)], idx_v)
        lax.fori_loop(0, CB // 16, recidx, 0, unroll=2)
        rows_dma = pltpu.async_copy(w_hbm.at[idxq_v], rows_v, sem)
        scale_dma = pltpu.async_copy(scale_hbm.at[idxs_v], scale_v, sem)
        rows_dma.wait()
        scale_dma.wait()
        lax.fori_loop(0, CB // 16, scalesel, 0, unroll=2)

        @pl.when(t >= 1)
        def _drain():
            pltpu.make_async_copy(
                out_hbm.at[pl.ds(0, D), pl.ds(0, CB)], out_v, osem).wait()

        lax.fori_loop(0, CB // 2, body, 0, unroll=2)
        pltpu.async_copy(
            out_v, out_hbm.at[pl.ds(lrow * D, D), pl.ds(bc * CB, CB)], osem)
        return carry

    lax.fori_loop(0, PER_W, item_loop, 0)
    pltpu.make_async_copy(
        out_hbm.at[pl.ds(0, D), pl.ds(0, CB)], out_v, osem).wait()


@jax.jit
def _run(idxt, weight, scaleq):
    mesh = plsc.VectorSubcoreMesh(core_axis_name="c", subcore_axis_name="s")
    f = functools.partial(
        pl.kernel,
        mesh=mesh,
        out_type=jax.ShapeDtypeStruct((L * D, B), jnp.float32),
        scratch_types=[
            pltpu.VMEM((CB,), jnp.int32),
            pltpu.VMEM((CB,), jnp.int32),
            pltpu.VMEM((CB,), jnp.int32),
            pltpu.VMEM((CB, 64), jnp.int8),
            pltpu.VMEM((CB, 16), jnp.float32),
            pltpu.VMEM((CB,), jnp.int32),
            pltpu.VMEM((CB,), jnp.float32),
            pltpu.VMEM((D, CB), jnp.float32),
            pltpu.SemaphoreType.DMA,
            pltpu.SemaphoreType.DMA,
        ],
        compiler_params=pltpu.CompilerParams(
            needs_layout_passes=False, use_tc_tiling_on_sc=False),
    )(_dequant_lookup)
    return f(idxt, weight, scaleq)


def kernel(input, weight, weight_scale):
    idxt = input.T.reshape(-1)
    wrec = weight.reshape(V // 4, 64)
    scaleq = weight_scale.reshape(V // 16, 16)
    out = _run(idxt, wrec, scaleq)
    return out.reshape(L, D, B).transpose(2, 0, 1)
